# baseline (speedup 1.0000x reference)
import jax
import jax.numpy as jnp
from jax import lax
from jax.experimental import pallas as pl
from jax.experimental.pallas import tpu as pltpu
from jax.experimental.pallas import tpu_sc as plsc

N_NODES = 50000
D = 128
E3 = 600000

NC, NS = 2, 16

CHUNK = 12544
NPAD = 4 * CHUNK
ROWS_PER_TILE = CHUNK // NS
DEGPAD = 51200
PAD_COL = 50200

E3P = 655360
EA_BATCHES = 160
SLAB = 2048
NSLABS = 20
GROUPS = SLAB // 16
KFLUSH = 64
STAGE = 80


def _deg_kernel(col_hbm, ew_hbm, deg_out, colv, ewv, zbuf, deg_sh, sem):
    c = lax.axis_index("c")
    s = lax.axis_index("s")
    wid = s * NC + c

    def _zb(i, carry):
        zbuf[pl.ds(i * 16, 16)] = jnp.zeros((16,), jnp.float32)
        return carry

    lax.fori_loop(0, 3200 // 16, _zb, 0)
    pltpu.sync_copy(zbuf, deg_sh.at[pl.ds(s * 3200, 3200)])
    plsc.subcore_barrier()

    pltpu.sync_copy(col_hbm.at[wid], colv)
    pltpu.sync_copy(ew_hbm.at[wid], ewv)

    def _scat(j, carry):
        pltpu.async_copy(ewv.at[j], deg_sh.at[colv.at[j]], sem, add=True)
        return carry

    lax.fori_loop(0, EA_BATCHES, _scat, 0)

    def _drainA(j, carry):
        pltpu.make_async_copy(ewv.at[j], deg_sh.at[colv.at[j]], sem).wait()
        return carry

    lax.fori_loop(0, EA_BATCHES, _drainA, 0)
    plsc.subcore_barrier()
    pltpu.sync_copy(deg_sh.at[pl.ds(s * 3200, 3200)],
                    deg_out.at[c, pl.ds(s * 3200, 3200)])


def _dinv(d0, d1):
    d = d0 + d1
    safe = jnp.where(d > 0, d, 1.0)
    return jnp.where(d > 0, lax.rsqrt(safe), 0.0)


def _mm_body(x_ref, w_ref, d0_ref, d1_ref, g_ref):
    dinv = _dinv(d0_ref[...], d1_ref[...])
    g_ref[...] = dinv * jnp.dot(x_ref[...], w_ref[...],
                                preferred_element_type=jnp.float32)


def _epi_body(a_ref, d0_ref, d1_ref, o_ref):
    dinv = _dinv(d0_ref[...], d1_ref[...])
    o_ref[...] = jnp.maximum(dinv * a_ref[...], 0.0)


def _agg_kernel(g_hbm, col_hbm, row_hbm, ew_hbm, out_hbm,
                colv2, rowv2, ewv2, scol, srow, sew,
                scol2, srow2, sew2, rows2, acc_sh, semg, sems,
                semsl0, semsl1):
    c = lax.axis_index("c")
    s = lax.axis_index("s")

    def _fire_slab(sb, bufi, semsl):
        pltpu.async_copy(col_hbm.at[s, sb], colv2.at[bufi], semsl)
        pltpu.async_copy(row_hbm.at[s, sb], rowv2.at[bufi], semsl)
        pltpu.async_copy(ew_hbm.at[s, sb], ewv2.at[bufi], semsl)

    def _wait_slab(sb, bufi, semsl):
        pltpu.make_async_copy(col_hbm.at[s, sb], colv2.at[bufi],
                              semsl).wait()
        pltpu.make_async_copy(row_hbm.at[s, sb], rowv2.at[bufi],
                              semsl).wait()
        pltpu.make_async_copy(ew_hbm.at[s, sb], ewv2.at[bufi], semsl).wait()

    def _zr(r, carry):
        for ch in range(8):
            rows2[0, r, pl.ds(ch * 16, 16)] = jnp.zeros((16,), jnp.float32)
        return carry

    for g in range(STAGE // 16):
        scol[pl.ds(g * 16, 16)] = jnp.zeros((16,), jnp.int32)
        srow[pl.ds(g * 16, 16)] = jnp.zeros((16,), jnp.int32)
        sew[pl.ds(g * 16, 16)] = jnp.zeros((16,), jnp.float32)

    def _scale(b):
        def body(gi, carry):
            w16 = sew2[b, pl.ds(gi * 16, 16)]
            for j in range(16):
                wj = w16.at[jnp.full((16,), j, jnp.int32)].get(
                    mode="promise_in_bounds")
                e = gi * 16 + j
                for ch in range(8):
                    rows2[b, e, pl.ds(ch * 16, 16)] = (
                        rows2[b, e, pl.ds(ch * 16, 16)] * wj)
            return carry

        lax.fori_loop(0, KFLUSH // 16, body, 0)

    def _wait_gather(b):
        pltpu.make_async_copy(g_hbm.at[srow2.at[b]], rows2.at[b], semg).wait()

    def _wait_scatter(b):
        pltpu.make_async_copy(rows2.at[b], acc_sh.at[scol2.at[b]],
                              sems).wait()

    def _flush(k, b):
        pb = 1 - b

        @pl.when(k >= 1)
        def _():
            _wait_gather(pb)
            _scale(pb)

            @pl.when(k >= 2)
            def _():
                _wait_scatter(b)

            pltpu.async_copy(rows2.at[pb], acc_sh.at[scol2.at[pb]], sems,
                             add=True)

        for g2 in range(KFLUSH // 16):
            scol2[b, pl.ds(g2 * 16, 16)] = scol[pl.ds(g2 * 16, 16)]
            srow2[b, pl.ds(g2 * 16, 16)] = srow[pl.ds(g2 * 16, 16)]
            sew2[b, pl.ds(g2 * 16, 16)] = sew[pl.ds(g2 * 16, 16)]
        pltpu.async_copy(g_hbm.at[srow2.at[b]], rows2.at[b], semg)

    def _drain(k, bl):
        pbl = 1 - bl
        _wait_gather(bl)
        _scale(bl)

        @pl.when(k >= 2)
        def _():
            _wait_scatter(pbl)

        pltpu.async_copy(rows2.at[bl], acc_sh.at[scol2.at[bl]], sems,
                         add=True)
        _wait_scatter(bl)

    for p in range(2):
        base = (c * 2 + p) * CHUNK

        lax.fori_loop(0, KFLUSH, _zr, 0)
        for kk in range(ROWS_PER_TILE // KFLUSH):
            pltpu.sync_copy(rows2.at[0],
                            acc_sh.at[pl.ds(s * ROWS_PER_TILE + kk * KFLUSH,
                                            KFLUSH)])
        rem = ROWS_PER_TILE % KFLUSH
        if rem:
            pltpu.sync_copy(
                rows2.at[0, pl.ds(0, rem)],
                acc_sh.at[pl.ds(s * ROWS_PER_TILE
                                + (ROWS_PER_TILE // KFLUSH) * KFLUSH, rem)])
        plsc.subcore_barrier()

        def _slab(sb, carry):
            cur = sb & 1

            @pl.when(sb + 1 < NSLABS)
            def _():
                @pl.when(cur == 0)
                def _():
                    _fire_slab(sb + 1, 1, semsl1)

                @pl.when(cur == 1)
                def _():
                    _fire_slab(sb + 1, 0, semsl0)

            @pl.when(cur == 0)
            def _():
                _wait_slab(sb, 0, semsl0)

            @pl.when(cur == 1)
            def _():
                _wait_slab(sb, 1, semsl1)

            def _group(gidx, carry):
                off, k = carry
                col16 = colv2[cur, pl.ds(gidx * 16, 16)]
                row16 = rowv2[cur, pl.ds(gidx * 16, 16)]
                w16 = ewv2[cur, pl.ds(gidx * 16, 16)]
                lcol = col16 - base
                m = (lcol >= 0) & (lcol < CHUNK)
                plsc.store_compressed(scol.at[pl.ds(off, 16)], lcol, mask=m)
                plsc.store_compressed(srow.at[pl.ds(off, 16)], row16, mask=m)
                plsc.store_compressed(sew.at[pl.ds(off, 16)], w16, mask=m)
                off = off + plsc.all_reduce_population_count(m)[0]
                do = off >= KFLUSH

                @pl.when(do)
                def _():
                    def _fl(b):
                        _flush(k, b)
                        vc = scol[pl.ds(KFLUSH, 16)]
                        scol[pl.ds(0, 16)] = vc
                        vr = srow[pl.ds(KFLUSH, 16)]
                        srow[pl.ds(0, 16)] = vr
                        vw = sew[pl.ds(KFLUSH, 16)]
                        sew[pl.ds(0, 16)] = vw

                    @pl.when((k & 1) == 0)
                    def _():
                        _fl(0)

                    @pl.when((k & 1) == 1)
                    def _():
                        _fl(1)

                return (jnp.where(do, off - KFLUSH, off),
                        jnp.where(do, k + 1, k))

            return lax.fori_loop(0, GROUPS, _group, carry)

        _fire_slab(0, 0, semsl0)
        off, k = lax.fori_loop(0, NSLABS, _slab,
                               (jnp.int32(0), jnp.int32(0)))

        iota16 = lax.iota(jnp.int32, 16)
        for g in range(STAGE // 16):
            lane = iota16 + g * 16
            cur = sew[pl.ds(g * 16, 16)]
            sew[pl.ds(g * 16, 16)] = jnp.where(lane < off, cur, 0.0)

        @pl.when((k & 1) == 0)
        def _():
            _flush(k, 0)
            _drain(k + 1, 0)

        @pl.when((k & 1) == 1)
        def _():
            _flush(k, 1)
            _drain(k + 1, 1)

        plsc.subcore_barrier()
        pltpu.sync_copy(acc_sh.at[pl.ds(s * ROWS_PER_TILE, ROWS_PER_TILE)],
                        out_hbm.at[pl.ds(base + s * ROWS_PER_TILE,
                                         ROWS_PER_TILE)])


def kernel(x, u_edge_index, u_edge_weight, v_edge_index, v_edge_weight,
           w_edge_index, w_edge_weight, W):
    n_pad = E3P - E3
    rows = jnp.concatenate([u_edge_index[0], v_edge_index[0], w_edge_index[0],
                            jnp.zeros((n_pad,), jnp.int32)])
    cols = jnp.concatenate([u_edge_index[1], v_edge_index[1], w_edge_index[1],
                            jnp.full((n_pad,), PAD_COL, jnp.int32)])
    ew = jnp.concatenate([u_edge_weight, v_edge_weight, w_edge_weight,
                          jnp.zeros((n_pad,), jnp.float32)])

    mesh = plsc.VectorSubcoreMesh(core_axis_name="c", subcore_axis_name="s")
    sc_params = pltpu.CompilerParams(needs_layout_passes=False)

    degp = pl.kernel(
        _deg_kernel,
        out_type=jax.ShapeDtypeStruct((NC, DEGPAD), jnp.float32),
        mesh=mesh,
        compiler_params=sc_params,
        scratch_types=[
            pltpu.VMEM((EA_BATCHES, 128), jnp.int32),
            pltpu.VMEM((EA_BATCHES, 128), jnp.float32),
            pltpu.VMEM((3200,), jnp.float32),
            pltpu.VMEM_SHARED((DEGPAD,), jnp.float32),
            pltpu.SemaphoreType.DMA,
        ],
    )(cols.reshape(NC * NS, EA_BATCHES, 128),
      ew.reshape(NC * NS, EA_BATCHES, 128))

    deg0 = degp[0].reshape(DEGPAD, 1)[:N_NODES]
    deg1 = degp[1].reshape(DEGPAD, 1)[:N_NODES]
    g = pl.pallas_call(
        _mm_body,
        grid=(25,),
        in_specs=[pl.BlockSpec((2000, D), lambda i: (i, 0)),
                  pl.BlockSpec((D, D), lambda i: (0, 0)),
                  pl.BlockSpec((2000, 1), lambda i: (i, 0)),
                  pl.BlockSpec((2000, 1), lambda i: (i, 0))],
        out_specs=pl.BlockSpec((2000, D), lambda i: (i, 0)),
        out_shape=jax.ShapeDtypeStruct((N_NODES, D), jnp.float32),
    )(x, W, deg0, deg1)

    out_acc = pl.kernel(
        _agg_kernel,
        out_type=jax.ShapeDtypeStruct((NPAD, D), jnp.float32),
        mesh=mesh,
        compiler_params=sc_params,
        scratch_types=[
            pltpu.VMEM((2, SLAB), jnp.int32),
            pltpu.VMEM((2, SLAB), jnp.int32),
            pltpu.VMEM((2, SLAB), jnp.float32),
            pltpu.VMEM((STAGE,), jnp.int32),
            pltpu.VMEM((STAGE,), jnp.int32),
            pltpu.VMEM((STAGE,), jnp.float32),
            pltpu.VMEM((2, KFLUSH), jnp.int32),
            pltpu.VMEM((2, KFLUSH), jnp.int32),
            pltpu.VMEM((2, KFLUSH), jnp.float32),
            pltpu.VMEM((2, KFLUSH, D), jnp.float32),
            pltpu.VMEM_SHARED((CHUNK, D), jnp.float32),
            pltpu.SemaphoreType.DMA,
            pltpu.SemaphoreType.DMA,
            pltpu.SemaphoreType.DMA,
            pltpu.SemaphoreType.DMA,
        ],
    )(g, cols.reshape(NS, NSLABS, SLAB), rows.reshape(NS, NSLABS, SLAB),
      ew.reshape(NS, NSLABS, SLAB))

    return pl.pallas_call(
        _epi_body,
        grid=(25,),
        in_specs=[pl.BlockSpec((2000, D), lambda i: (i, 0)),
                  pl.BlockSpec((2000, 1), lambda i: (i, 0)),
                  pl.BlockSpec((2000, 1), lambda i: (i, 0))],
        out_specs=pl.BlockSpec((2000, D), lambda i: (i, 0)),
        out_shape=jax.ShapeDtypeStruct((N_NODES, D), jnp.float32),
    )(out_acc, deg0, deg1)


# speedup vs baseline: 1.0629x; 1.0629x over previous
import jax
import jax.numpy as jnp
from jax import lax
from jax.experimental import pallas as pl
from jax.experimental.pallas import tpu as pltpu
from jax.experimental.pallas import tpu_sc as plsc

N_NODES = 50000
D = 128
E3 = 600000

NC, NS = 2, 16

CHUNK = 12544
NPAD = 4 * CHUNK
ROWS_PER_TILE = CHUNK // NS
DEGPAD = 51200
PAD_COL = 50200

E3P = 655360
EA_BATCHES = 160
SLAB = 2048
NSLABS = 20
GROUPS = SLAB // 16
KFLUSH = 80
STAGE = 96


def _deg_kernel(col_hbm, ew_hbm, deg_out, colv, ewv, zbuf, deg_sh, sem):
    c = lax.axis_index("c")
    s = lax.axis_index("s")
    wid = s * NC + c

    def _zb(i, carry):
        zbuf[pl.ds(i * 16, 16)] = jnp.zeros((16,), jnp.float32)
        return carry

    lax.fori_loop(0, 3200 // 16, _zb, 0)
    pltpu.sync_copy(zbuf, deg_sh.at[pl.ds(s * 3200, 3200)])
    plsc.subcore_barrier()

    pltpu.sync_copy(col_hbm.at[wid], colv)
    pltpu.sync_copy(ew_hbm.at[wid], ewv)

    def _scat(j, carry):
        pltpu.async_copy(ewv.at[j], deg_sh.at[colv.at[j]], sem, add=True)
        return carry

    lax.fori_loop(0, EA_BATCHES, _scat, 0)

    def _drainA(j, carry):
        pltpu.make_async_copy(ewv.at[j], deg_sh.at[colv.at[j]], sem).wait()
        return carry

    lax.fori_loop(0, EA_BATCHES, _drainA, 0)
    plsc.subcore_barrier()
    pltpu.sync_copy(deg_sh.at[pl.ds(s * 3200, 3200)],
                    deg_out.at[c, pl.ds(s * 3200, 3200)])


def _dinv(d0, d1):
    d = d0 + d1
    safe = jnp.where(d > 0, d, 1.0)
    return jnp.where(d > 0, lax.rsqrt(safe), 0.0)


def _mm_body(x_ref, w_ref, d0_ref, d1_ref, g_ref):
    dinv = _dinv(d0_ref[...], d1_ref[...])
    g_ref[...] = dinv * jnp.dot(x_ref[...], w_ref[...],
                                preferred_element_type=jnp.float32)


def _epi_body(a_ref, d0_ref, d1_ref, o_ref):
    dinv = _dinv(d0_ref[...], d1_ref[...])
    o_ref[...] = jnp.maximum(dinv * a_ref[...], 0.0)


def _agg_kernel(g_hbm, pk_hbm, ew_hbm, out_hbm,
                pkv2, ewv2, spk, sew,
                scol2, srow2, sew2, rows2, acc_sh, semg, sems,
                semsl0, semsl1):
    c = lax.axis_index("c")
    s = lax.axis_index("s")

    def _fire_slab(sb, bufi, semsl):
        pltpu.async_copy(pk_hbm.at[s, sb], pkv2.at[bufi], semsl)
        pltpu.async_copy(ew_hbm.at[s, sb], ewv2.at[bufi], semsl)

    def _wait_slab(sb, bufi, semsl):
        pltpu.make_async_copy(pk_hbm.at[s, sb], pkv2.at[bufi], semsl).wait()
        pltpu.make_async_copy(ew_hbm.at[s, sb], ewv2.at[bufi], semsl).wait()

    def _zr(r, carry):
        for ch in range(8):
            rows2[0, r, pl.ds(ch * 16, 16)] = jnp.zeros((16,), jnp.float32)
        return carry

    for g in range(STAGE // 16):
        spk[pl.ds(g * 16, 16)] = jnp.zeros((16,), jnp.uint32)
        sew[pl.ds(g * 16, 16)] = jnp.zeros((16,), jnp.float32)

    def _scale(b):
        def body(gi, carry):
            w16 = sew2[b, pl.ds(gi * 16, 16)]
            for j in range(16):
                wj = w16.at[jnp.full((16,), j, jnp.int32)].get(
                    mode="promise_in_bounds")
                e = gi * 16 + j
                for ch in range(8):
                    rows2[b, e, pl.ds(ch * 16, 16)] = (
                        rows2[b, e, pl.ds(ch * 16, 16)] * wj)
            return carry

        lax.fori_loop(0, KFLUSH // 16, body, 0)

    def _wait_gather(b):
        pltpu.make_async_copy(g_hbm.at[srow2.at[b]], rows2.at[b], semg).wait()

    def _wait_scatter(b):
        pltpu.make_async_copy(rows2.at[b], acc_sh.at[scol2.at[b]],
                              sems).wait()

    def _flush(k, b, base):
        pb = 1 - b

        @pl.when(k >= 1)
        def _():
            _wait_gather(pb)
            _scale(pb)

            @pl.when(k >= 2)
            def _():
                _wait_scatter(b)

            pltpu.async_copy(rows2.at[pb], acc_sh.at[scol2.at[pb]], sems,
                             add=True)

        for g2 in range(KFLUSH // 16):
            pk = spk[pl.ds(g2 * 16, 16)]
            lcol = (pk & jnp.uint32(0xFFFF)).astype(jnp.int32) - base
            ok = (lcol >= 0) & (lcol < CHUNK)
            scol2[b, pl.ds(g2 * 16, 16)] = jnp.where(ok, lcol, 0)
            srow2[b, pl.ds(g2 * 16, 16)] = (pk >> jnp.uint32(16)).astype(
                jnp.int32)
            sew2[b, pl.ds(g2 * 16, 16)] = sew[pl.ds(g2 * 16, 16)]
        pltpu.async_copy(g_hbm.at[srow2.at[b]], rows2.at[b], semg)

    def _drain(k, bl):
        pbl = 1 - bl
        _wait_gather(bl)
        _scale(bl)

        @pl.when(k >= 2)
        def _():
            _wait_scatter(pbl)

        pltpu.async_copy(rows2.at[bl], acc_sh.at[scol2.at[bl]], sems,
                         add=True)
        _wait_scatter(bl)

    for p in range(2):
        base = (c * 2 + p) * CHUNK
        base_u = base.astype(jnp.uint32)

        lax.fori_loop(0, KFLUSH, _zr, 0)
        for kk in range(ROWS_PER_TILE // KFLUSH):
            pltpu.sync_copy(rows2.at[0],
                            acc_sh.at[pl.ds(s * ROWS_PER_TILE + kk * KFLUSH,
                                            KFLUSH)])
        rem = ROWS_PER_TILE % KFLUSH
        if rem:
            pltpu.sync_copy(
                rows2.at[0, pl.ds(0, rem)],
                acc_sh.at[pl.ds(s * ROWS_PER_TILE
                                + (ROWS_PER_TILE // KFLUSH) * KFLUSH, rem)])
        plsc.subcore_barrier()

        def _slab(sb, carry):
            cur = sb & 1

            @pl.when(sb + 1 < NSLABS)
            def _():
                @pl.when(cur == 0)
                def _():
                    _fire_slab(sb + 1, 1, semsl1)

                @pl.when(cur == 1)
                def _():
                    _fire_slab(sb + 1, 0, semsl0)

            @pl.when(cur == 0)
            def _():
                _wait_slab(sb, 0, semsl0)

            @pl.when(cur == 1)
            def _():
                _wait_slab(sb, 1, semsl1)

            def _group(gidx, carry):
                off, k = carry
                p16 = pkv2[cur, pl.ds(gidx * 16, 16)]
                w16 = ewv2[cur, pl.ds(gidx * 16, 16)]
                col16 = p16 & jnp.uint32(0xFFFF)
                m = (col16 >= base_u) & (col16 < base_u + CHUNK)
                plsc.store_compressed(spk.at[pl.ds(off, 16)], p16, mask=m)
                plsc.store_compressed(sew.at[pl.ds(off, 16)], w16, mask=m)
                off = off + plsc.all_reduce_population_count(m)[0]
                do = off >= KFLUSH

                @pl.when(do)
                def _():
                    def _fl(b):
                        _flush(k, b, base)
                        vc = spk[pl.ds(KFLUSH, 16)]
                        spk[pl.ds(0, 16)] = vc
                        vw = sew[pl.ds(KFLUSH, 16)]
                        sew[pl.ds(0, 16)] = vw

                    @pl.when((k & 1) == 0)
                    def _():
                        _fl(0)

                    @pl.when((k & 1) == 1)
                    def _():
                        _fl(1)

                return (jnp.where(do, off - KFLUSH, off),
                        jnp.where(do, k + 1, k))

            return lax.fori_loop(0, GROUPS, _group, carry)

        _fire_slab(0, 0, semsl0)
        off, k = lax.fori_loop(0, NSLABS, _slab,
                               (jnp.int32(0), jnp.int32(0)))

        iota16 = lax.iota(jnp.int32, 16)
        for g in range(STAGE // 16):
            lane = iota16 + g * 16
            cur = sew[pl.ds(g * 16, 16)]
            sew[pl.ds(g * 16, 16)] = jnp.where(lane < off, cur, 0.0)

        @pl.when((k & 1) == 0)
        def _():
            _flush(k, 0, base)
            _drain(k + 1, 0)

        @pl.when((k & 1) == 1)
        def _():
            _flush(k, 1, base)
            _drain(k + 1, 1)

        plsc.subcore_barrier()
        pltpu.sync_copy(acc_sh.at[pl.ds(s * ROWS_PER_TILE, ROWS_PER_TILE)],
                        out_hbm.at[pl.ds(base + s * ROWS_PER_TILE,
                                         ROWS_PER_TILE)])


def kernel(x, u_edge_index, u_edge_weight, v_edge_index, v_edge_weight,
           w_edge_index, w_edge_weight, W):
    n_pad = E3P - E3
    rows = jnp.concatenate([u_edge_index[0], v_edge_index[0], w_edge_index[0],
                            jnp.zeros((n_pad,), jnp.int32)])
    cols = jnp.concatenate([u_edge_index[1], v_edge_index[1], w_edge_index[1],
                            jnp.full((n_pad,), PAD_COL, jnp.int32)])
    ew = jnp.concatenate([u_edge_weight, v_edge_weight, w_edge_weight,
                          jnp.zeros((n_pad,), jnp.float32)])
    packed = ((rows.astype(jnp.uint32) << jnp.uint32(16))
              | cols.astype(jnp.uint32))

    mesh = plsc.VectorSubcoreMesh(core_axis_name="c", subcore_axis_name="s")
    sc_params = pltpu.CompilerParams(needs_layout_passes=False)

    degp = pl.kernel(
        _deg_kernel,
        out_type=jax.ShapeDtypeStruct((NC, DEGPAD), jnp.float32),
        mesh=mesh,
        compiler_params=sc_params,
        scratch_types=[
            pltpu.VMEM((EA_BATCHES, 128), jnp.int32),
            pltpu.VMEM((EA_BATCHES, 128), jnp.float32),
            pltpu.VMEM((3200,), jnp.float32),
            pltpu.VMEM_SHARED((DEGPAD,), jnp.float32),
            pltpu.SemaphoreType.DMA,
        ],
    )(cols.reshape(NC * NS, EA_BATCHES, 128),
      ew.reshape(NC * NS, EA_BATCHES, 128))

    deg0 = degp[0].reshape(DEGPAD, 1)[:N_NODES]
    deg1 = degp[1].reshape(DEGPAD, 1)[:N_NODES]
    g = pl.pallas_call(
        _mm_body,
        grid=(25,),
        in_specs=[pl.BlockSpec((2000, D), lambda i: (i, 0)),
                  pl.BlockSpec((D, D), lambda i: (0, 0)),
                  pl.BlockSpec((2000, 1), lambda i: (i, 0)),
                  pl.BlockSpec((2000, 1), lambda i: (i, 0))],
        out_specs=pl.BlockSpec((2000, D), lambda i: (i, 0)),
        out_shape=jax.ShapeDtypeStruct((N_NODES, D), jnp.float32),
    )(x, W, deg0, deg1)

    out_acc = pl.kernel(
        _agg_kernel,
        out_type=jax.ShapeDtypeStruct((NPAD, D), jnp.float32),
        mesh=mesh,
        compiler_params=sc_params,
        scratch_types=[
            pltpu.VMEM((2, SLAB), jnp.uint32),
            pltpu.VMEM((2, SLAB), jnp.float32),
            pltpu.VMEM((STAGE,), jnp.uint32),
            pltpu.VMEM((STAGE,), jnp.float32),
            pltpu.VMEM((2, KFLUSH), jnp.int32),
            pltpu.VMEM((2, KFLUSH), jnp.int32),
            pltpu.VMEM((2, KFLUSH), jnp.float32),
            pltpu.VMEM((2, KFLUSH, D), jnp.float32),
            pltpu.VMEM_SHARED((CHUNK, D), jnp.float32),
            pltpu.SemaphoreType.DMA,
            pltpu.SemaphoreType.DMA,
            pltpu.SemaphoreType.DMA,
            pltpu.SemaphoreType.DMA,
        ],
    )(g, packed.reshape(NS, NSLABS, SLAB), ew.reshape(NS, NSLABS, SLAB))

    return pl.pallas_call(
        _epi_body,
        grid=(25,),
        in_specs=[pl.BlockSpec((2000, D), lambda i: (i, 0)),
                  pl.BlockSpec((2000, 1), lambda i: (i, 0)),
                  pl.BlockSpec((2000, 1), lambda i: (i, 0))],
        out_specs=pl.BlockSpec((2000, D), lambda i: (i, 0)),
        out_shape=jax.ShapeDtypeStruct((N_NODES, D), jnp.float32),
    )(out_acc, deg0, deg1)
